# baseline (speedup 1.0000x reference)
import functools

import jax
import jax.numpy as jnp
from jax import lax
from jax.experimental import pallas as pl
from jax.experimental.pallas import tpu as pltpu
from jax.experimental.pallas import tpu_sc as plsc

HEADS = 8
POINTS = 4
HD = 32
BQ = 256


def _stage_a_body(qe_ref, ctrl_ref, wq_ref, bq_ref, wso_ref, bso_ref,
                  waw_ref, baw_ref, pc_ref, sp_ref,
                  q_out_ref, idx_out_ref, w_out_ref):
    f32 = jnp.float32
    bf16 = jnp.bfloat16
    q = jnp.dot(qe_ref[...].astype(bf16), wq_ref[...].astype(bf16),
                preferred_element_type=f32) + bq_ref[...]
    q_out_ref[...] = q

    qb = q.astype(bf16)
    so = jnp.dot(qb, wso_ref[...].astype(bf16), preferred_element_type=f32) + bso_ref[...]
    awl = jnp.dot(qb, waw_ref[...].astype(bf16), preferred_element_type=f32) + baw_ref[...]
    awl = awl - jnp.max(awl, axis=1, keepdims=True)
    e = jnp.exp(awl)
    col = lax.broadcasted_iota(jnp.int32, (HEADS * POINTS, HEADS * POINTS), 0)
    row = lax.broadcasted_iota(jnp.int32, (HEADS * POINTS, HEADS * POINTS), 1)
    gmask = (col // POINTS == row // POINTS).astype(f32)
    gsum = jnp.dot(e, gmask, preferred_element_type=f32, precision=jax.lax.Precision.HIGHEST)
    aw = e / gsum

    pc0, pc1, pc3, pc4 = pc_ref[0], pc_ref[1], pc_ref[3], pc_ref[4]
    cx = ctrl_ref[:, 0:1].astype(bf16).astype(f32)
    cy = ctrl_ref[:, 1:2].astype(bf16).astype(f32)
    rx = jnp.clip((cx - pc0) / (pc3 - pc0), 0.01, 0.99)
    ry = jnp.clip((cy - pc1) / (pc4 - pc1), 0.01, 0.99)

    wn = sp_ref[0, 1].astype(f32)
    hn = sp_ref[0, 0].astype(f32)
    hs = sp_ref[0, 0]
    ws = sp_ref[0, 1]
    slx = rx + so[:, 0:32] / wn
    sly = ry + so[:, 32:64] / hn
    gx = (2.0 * slx - 1.0 + 1.0) * wn / 2.0 - 0.5
    gy = (2.0 * sly - 1.0 + 1.0) * hn / 2.0 - 0.5
    x0 = jnp.floor(gx)
    y0 = jnp.floor(gy)
    wx1 = gx - x0
    wx0 = 1.0 - wx1
    wy1 = gy - y0
    wy0 = 1.0 - wy1

    hcol = lax.broadcasted_iota(jnp.int32, (BQ, HEADS * POINTS), 1) // POINTS
    hg = hcol // 4
    wf = ws.astype(f32)
    hf = hs.astype(f32)
    npair = ws // 2 + 1

    def cweight(xo, yo, wx, wy):
        ix = x0 + xo
        iy = y0 + yo
        valid = (ix >= 0.0) & (ix <= wf - 1.0) & (iy >= 0.0) & (iy <= hf - 1.0)
        return jnp.where(valid, wx * wy * aw, 0.0)

    g = (jnp.clip(x0, -1.0, wf - 1.0) + 1.0).astype(jnp.int32)
    i_p = g // 2
    dxx = g - 2 * i_p
    base = (dxx * 2 + hg) * hs * npair + i_p
    yc0 = jnp.clip(y0, 0.0, hf - 1.0).astype(jnp.int32)
    yc1 = jnp.clip(y0 + 1.0, 0.0, hf - 1.0).astype(jnp.int32)
    t_top = base + yc0 * npair
    t_bot = base + yc1 * npair
    idx_out_ref[...] = jnp.concatenate([t_top, t_bot], axis=1)
    w_out_ref[...] = jnp.concatenate(
        [cweight(0.0, 0.0, wx0, wy0), cweight(1.0, 0.0, wx1, wy0),
         cweight(0.0, 1.0, wx0, wy1), cweight(1.0, 1.0, wx1, wy1)], axis=1)


def _stage_a(qe, ctrl8, wq, bq, wso_r, bso_r, waw, baw, pc, sp):
    nq, d = qe.shape
    grid = (nq // BQ,)
    return pl.pallas_call(
        _stage_a_body,
        grid=grid,
        in_specs=[
            pl.BlockSpec((BQ, d), lambda i: (i, 0)),
            pl.BlockSpec((BQ, 8), lambda i: (i, 0)),
            pl.BlockSpec((d, d), lambda i: (0, 0)),
            pl.BlockSpec((1, d), lambda i: (0, 0)),
            pl.BlockSpec((d, 2 * HEADS * POINTS), lambda i: (0, 0)),
            pl.BlockSpec((1, 2 * HEADS * POINTS), lambda i: (0, 0)),
            pl.BlockSpec((d, HEADS * POINTS), lambda i: (0, 0)),
            pl.BlockSpec((1, HEADS * POINTS), lambda i: (0, 0)),
            pl.BlockSpec(memory_space=pltpu.SMEM),
            pl.BlockSpec(memory_space=pltpu.SMEM),
        ],
        out_specs=[
            pl.BlockSpec((BQ, d), lambda i: (i, 0)),
            pl.BlockSpec((BQ, 2 * HEADS * POINTS), lambda i: (i, 0)),
            pl.BlockSpec((BQ, 4 * HEADS * POINTS), lambda i: (i, 0)),
        ],
        out_shape=[
            jax.ShapeDtypeStruct((nq, d), jnp.float32),
            jax.ShapeDtypeStruct((nq, 2 * HEADS * POINTS), jnp.int32),
            jax.ShapeDtypeStruct((nq, 4 * HEADS * POINTS), jnp.float32),
        ],
    )(qe, ctrl8, wq, bq, wso_r, bso_r, waw, baw, pc, sp)


_YB = 8


def _value_body(bev_ref, wv_ref, bv_ref, out_ref, *, w):
    wv = wv_ref[...].astype(jnp.bfloat16)
    bv = bv_ref[...]
    c = bev_ref.shape[0]
    np_ = w // 2 + 1
    npx = _YB * w
    x = bev_ref[...].reshape(c, npx).astype(jnp.bfloat16)
    r = lax.dot_general(x, wv, (((0,), (0,)), ((), ())),
                        preferred_element_type=jnp.float32) + bv
    rs = r.reshape(npx, HEADS, 2, 16)
    af = rs[:, :, 0, :].astype(jnp.bfloat16).astype(jnp.float32)
    bf_ = rs[:, :, 1, :].astype(jnp.bfloat16).astype(jnp.float32)
    ai = lax.bitcast_convert_type(af, jnp.int32)
    bi = lax.bitcast_convert_type(bf_, jnp.int32)
    packed = lax.shift_right_logical(ai, 16) | (bi & jnp.int32(-65536))
    rp = lax.bitcast_convert_type(packed, jnp.float32).reshape(npx, 128)
    odd_all = jnp.concatenate([rp[0:1], rp, rp[-1:]], axis=0)
    for yy in range(_YB):
        ev = rp[yy * w:(yy + 1) * w].reshape(w // 2, 2, 2, 64)
        ev = ev.transpose(0, 2, 1, 3).reshape(w // 2, 2, 128)
        od = odd_all[yy * w:yy * w + w + 2].reshape(np_, 2, 2, 64)
        od = od.transpose(0, 2, 1, 3).reshape(np_, 2, 128)
        for hg in range(2):
            out_ref[0, hg, yy, :] = od[:, hg, :]
            out_ref[1, hg, yy, 0:w // 2] = ev[:, hg, :]
            out_ref[1, hg, yy, w // 2] = ev[w // 2 - 1, hg, :]


def _value_project(bev3, wv, bv, h, w):
    c = bev3.shape[0]
    np_ = w // 2 + 1
    grid = (h // _YB,)
    return pl.pallas_call(
        functools.partial(_value_body, w=w),
        grid=grid,
        in_specs=[
            pl.BlockSpec((c, _YB, w), lambda i: (0, i, 0)),
            pl.BlockSpec((c, c), lambda i: (0, 0)),
            pl.BlockSpec((1, c), lambda i: (0, 0)),
        ],
        out_specs=pl.BlockSpec((2, 2, _YB, np_, 128),
                               lambda i: (0, 0, i, 0, 0)),
        out_shape=jax.ShapeDtypeStruct((2, 2, h, np_, 128), jnp.float32),
    )(bev3, wv, bv)


def _stage_c_body(ms_ref, q_ref, wmo_ref, bmo_ref, wo_ref, bo_ref, out_ref):
    f32 = jnp.float32
    bf16 = jnp.bfloat16
    h1 = jnp.dot(ms_ref[...].astype(bf16), wmo_ref[...].astype(bf16),
                 preferred_element_type=f32)
    h1 = h1 + bmo_ref[...] + q_ref[...]
    out_ref[...] = jnp.dot(h1.astype(bf16), wo_ref[...].astype(bf16),
                           preferred_element_type=f32) + bo_ref[...]


def _stage_c(msda, q, wmo, bmo, wo, bo):
    nq, d = msda.shape
    grid = (nq // BQ,)
    return pl.pallas_call(
        _stage_c_body,
        grid=grid,
        in_specs=[
            pl.BlockSpec((BQ, d), lambda i: (i, 0)),
            pl.BlockSpec((BQ, d), lambda i: (i, 0)),
            pl.BlockSpec((d, d), lambda i: (0, 0)),
            pl.BlockSpec((1, d), lambda i: (0, 0)),
            pl.BlockSpec((d, d), lambda i: (0, 0)),
            pl.BlockSpec((1, d), lambda i: (0, 0)),
        ],
        out_specs=pl.BlockSpec((BQ, d), lambda i: (i, 0)),
        out_shape=jax.ShapeDtypeStruct((nq, d), jnp.float32),
    )(msda, q, wmo, bmo, wo, bo)


_NW = 32
_PPQ = 2 * HEADS * POINTS
_WPQ = 4 * HEADS * POINTS


def _sc_gather_combine(table, idx_flat, w_flat, nq):
    qpw = nq // _NW
    mesh = plsc.VectorSubcoreMesh(
        core_axis_name="c", subcore_axis_name="s", num_cores=2, num_subcores=16)

    chq = 2
    chr_ = chq * _PPQ
    nch = qpw // chq

    @functools.partial(
        pl.kernel,
        out_type=jax.ShapeDtypeStruct((nq, HEADS * HD), jnp.float32),
        mesh=mesh,
        scratch_types=[
            pltpu.VMEM((qpw * _PPQ,), jnp.int32),
            pltpu.VMEM((qpw * _WPQ,), jnp.float32),
            pltpu.VMEM((chr_, 128), jnp.float32),
            pltpu.VMEM((chr_, 128), jnp.float32),
            pltpu.VMEM((qpw, HEADS * HD), jnp.float32),
            pltpu.SemaphoreType.DMA,
            pltpu.SemaphoreType.DMA,
        ],
    )
    def k(table_hbm, idx_hbm, w_hbm, out_hbm, idx_v, w_v, rows0_v, rows1_v,
          out_v, sem0, sem1):
        wid = lax.axis_index("s") * 2 + lax.axis_index("c")
        pltpu.sync_copy(idx_hbm.at[pl.ds(wid * qpw * _PPQ, qpw * _PPQ)], idx_v)
        pltpu.sync_copy(w_hbm.at[pl.ds(wid * qpw * _WPQ, qpw * _WPQ)], w_v)

        def start(ch, buf, sem):
            pltpu.async_copy(
                table_hbm.at[idx_v.at[pl.ds(ch * chr_, chr_)]], buf, sem)

        def waitbuf(buf, sem):
            pltpu.make_async_copy(table_hbm.at[pl.ds(0, chr_)], buf, sem).wait()

        def combine(ch, buf):
            for qq in range(chq):
                wb = (ch * chq + qq) * _WPQ
                wvs = [w_v[pl.ds(wb + k * 16, 16)] for k in range(_WPQ // 16)]
                for h in range(HEADS):
                    acc0 = jnp.zeros((16,), jnp.float32)
                    acc1 = jnp.zeros((16,), jnp.float32)
                    ho = (h % 4) * 16
                    for c2 in range(2):
                        for p in range(POINTS):
                            pr = qq * _PPQ + c2 * HEADS * POINTS + h * POINTS + p
                            for slot in range(2):
                                rw = (c2 * 2 + slot) * HEADS * POINTS + h * POINTS + p
                                wgt = wvs[rw // 16][rw % 16]
                                u = lax.bitcast_convert_type(
                                    buf[pr, pl.ds(slot * 64 + ho, 16)], jnp.int32)
                                lo = lax.bitcast_convert_type(
                                    lax.shift_left(u, 16), jnp.float32)
                                hi = lax.bitcast_convert_type(
                                    u & jnp.int32(-65536), jnp.float32)
                                acc0 = acc0 + wgt * lo
                                acc1 = acc1 + wgt * hi
                    o = ch * chq + qq
                    out_v[o, pl.ds(h * HD, 16)] = acc0
                    out_v[o, pl.ds(h * HD + 16, 16)] = acc1

        start(0, rows0_v, sem0)

        def body(g, carry):
            start(2 * g + 1, rows1_v, sem1)
            waitbuf(rows0_v, sem0)
            combine(2 * g, rows0_v)

            @pl.when(g < nch // 2 - 1)
            def _():
                start(2 * g + 2, rows0_v, sem0)

            waitbuf(rows1_v, sem1)
            combine(2 * g + 1, rows1_v)
            return carry

        lax.fori_loop(0, nch // 2, body, 0)
        pltpu.sync_copy(out_v, out_hbm.at[pl.ds(wid * qpw, qpw)])

    return k(table, idx_flat, w_flat)


def kernel(query_embed, ctrl_points, bev_features, pc_range, Wq, bq, Wso, bso,
           Waw, baw, Wv, bv, Wmo, bmo, Wo, bo, spatial_shapes):
    b, nq, d = query_embed.shape
    _, c, h, w = bev_features.shape
    qe = query_embed.reshape(b * nq, d)
    ctrl8 = ctrl_points.reshape(b * nq, 8)
    wso_r = Wso.reshape(d, HEADS, POINTS, 2).transpose(0, 3, 1, 2).reshape(d, 2 * HEADS * POINTS)
    bso_r = bso.reshape(HEADS, POINTS, 2).transpose(2, 0, 1).reshape(1, 2 * HEADS * POINTS)

    qp, idx128, w128 = _stage_a(
        qe, ctrl8, Wq, bq.reshape(1, d), wso_r, bso_r,
        Waw, baw.reshape(1, HEADS * POINTS), pc_range, spatial_shapes)

    value = _value_project(bev_features.reshape(c, h, w), Wv, bv.reshape(1, d), h, w)
    table = value.reshape(2 * 2 * h * (w // 2 + 1), 128)

    msda = _sc_gather_combine(table, idx128.reshape(-1), w128.reshape(-1), b * nq)

    out = _stage_c(msda, qp, Wmo, bmo.reshape(1, d),
                   Wo, bo.reshape(1, d))
    return out.reshape(b, nq, d)


# speedup vs baseline: 2.6496x; 2.6496x over previous
import functools

import jax
import jax.numpy as jnp
from jax import lax
from jax.experimental import pallas as pl
from jax.experimental.pallas import tpu as pltpu
from jax.experimental.pallas import tpu_sc as plsc

HEADS = 8
POINTS = 4
HD = 32
BQ = 256


def _stage_a_body(qe_ref, ctrl_ref, wq_ref, bq_ref, wso_ref, bso_ref,
                  waw_ref, baw_ref, pc_ref, sp_ref,
                  q_out_ref, idx_out_ref, w_out_ref):
    f32 = jnp.float32
    bf16 = jnp.bfloat16
    q = jnp.dot(qe_ref[...].astype(bf16), wq_ref[...].astype(bf16),
                preferred_element_type=f32) + bq_ref[...]
    q_out_ref[...] = q

    qb = q.astype(bf16)
    so = jnp.dot(qb, wso_ref[...].astype(bf16), preferred_element_type=f32) + bso_ref[...]
    awl = jnp.dot(qb, waw_ref[...].astype(bf16), preferred_element_type=f32) + baw_ref[...]
    awl = awl - jnp.max(awl, axis=1, keepdims=True)
    e = jnp.exp(awl)
    col = lax.broadcasted_iota(jnp.int32, (HEADS * POINTS, HEADS * POINTS), 0)
    row = lax.broadcasted_iota(jnp.int32, (HEADS * POINTS, HEADS * POINTS), 1)
    gmask = (col // POINTS == row // POINTS).astype(f32)
    gsum = jnp.dot(e, gmask, preferred_element_type=f32, precision=jax.lax.Precision.HIGHEST)
    aw = e / gsum

    pc0, pc1, pc3, pc4 = pc_ref[0], pc_ref[1], pc_ref[3], pc_ref[4]
    cx = ctrl_ref[:, 0:1].astype(bf16).astype(f32)
    cy = ctrl_ref[:, 1:2].astype(bf16).astype(f32)
    rx = jnp.clip((cx - pc0) / (pc3 - pc0), 0.01, 0.99)
    ry = jnp.clip((cy - pc1) / (pc4 - pc1), 0.01, 0.99)

    wn = sp_ref[0, 1].astype(f32)
    hn = sp_ref[0, 0].astype(f32)
    hs = sp_ref[0, 0]
    ws = sp_ref[0, 1]
    slx = rx + so[:, 0:32] / wn
    sly = ry + so[:, 32:64] / hn
    gx = (2.0 * slx - 1.0 + 1.0) * wn / 2.0 - 0.5
    gy = (2.0 * sly - 1.0 + 1.0) * hn / 2.0 - 0.5
    x0 = jnp.floor(gx)
    y0 = jnp.floor(gy)
    wx1 = gx - x0
    wx0 = 1.0 - wx1
    wy1 = gy - y0
    wy0 = 1.0 - wy1

    hcol = lax.broadcasted_iota(jnp.int32, (BQ, HEADS * POINTS), 1) // POINTS
    hg = hcol // 4
    wf = ws.astype(f32)
    hf = hs.astype(f32)
    npair = ws // 2 + 1

    def cweight(xo, yo, wx, wy):
        ix = x0 + xo
        iy = y0 + yo
        valid = (ix >= 0.0) & (ix <= wf - 1.0) & (iy >= 0.0) & (iy <= hf - 1.0)
        return jnp.where(valid, wx * wy * aw, 0.0)

    g = (jnp.clip(x0, -1.0, wf - 1.0) + 1.0).astype(jnp.int32)
    i_p = g // 2
    dxx = g - 2 * i_p
    base = (dxx * 2 + hg) * hs * npair + i_p
    yc0 = jnp.clip(y0, 0.0, hf - 1.0).astype(jnp.int32)
    yc1 = jnp.clip(y0 + 1.0, 0.0, hf - 1.0).astype(jnp.int32)
    t_top = base + yc0 * npair
    t_bot = base + yc1 * npair
    idx_out_ref[...] = jnp.concatenate([t_top, t_bot], axis=1)
    w_out_ref[...] = jnp.concatenate(
        [cweight(0.0, 0.0, wx0, wy0), cweight(1.0, 0.0, wx1, wy0),
         cweight(0.0, 1.0, wx0, wy1), cweight(1.0, 1.0, wx1, wy1)], axis=1)


def _stage_a(qe, ctrl8, wq, bq, wso_r, bso_r, waw, baw, pc, sp):
    nq, d = qe.shape
    grid = (nq // BQ,)
    return pl.pallas_call(
        _stage_a_body,
        grid=grid,
        in_specs=[
            pl.BlockSpec((BQ, d), lambda i: (i, 0)),
            pl.BlockSpec((BQ, 8), lambda i: (i, 0)),
            pl.BlockSpec((d, d), lambda i: (0, 0)),
            pl.BlockSpec((1, d), lambda i: (0, 0)),
            pl.BlockSpec((d, 2 * HEADS * POINTS), lambda i: (0, 0)),
            pl.BlockSpec((1, 2 * HEADS * POINTS), lambda i: (0, 0)),
            pl.BlockSpec((d, HEADS * POINTS), lambda i: (0, 0)),
            pl.BlockSpec((1, HEADS * POINTS), lambda i: (0, 0)),
            pl.BlockSpec(memory_space=pltpu.SMEM),
            pl.BlockSpec(memory_space=pltpu.SMEM),
        ],
        out_specs=[
            pl.BlockSpec((BQ, d), lambda i: (i, 0)),
            pl.BlockSpec((BQ, 2 * HEADS * POINTS), lambda i: (i, 0)),
            pl.BlockSpec((BQ, 4 * HEADS * POINTS), lambda i: (i, 0)),
        ],
        out_shape=[
            jax.ShapeDtypeStruct((nq, d), jnp.float32),
            jax.ShapeDtypeStruct((nq, 2 * HEADS * POINTS), jnp.int32),
            jax.ShapeDtypeStruct((nq, 4 * HEADS * POINTS), jnp.float32),
        ],
    )(qe, ctrl8, wq, bq, wso_r, bso_r, waw, baw, pc, sp)


_YB = 8


def _value_body(bev_ref, wv_ref, bv_ref, out_ref, *, w):
    wv = wv_ref[...].astype(jnp.bfloat16)
    bv = bv_ref[...]
    c = bev_ref.shape[0]
    np_ = w // 2 + 1
    npx = _YB * w
    x = bev_ref[...].reshape(c, npx).astype(jnp.bfloat16)
    r = lax.dot_general(x, wv, (((0,), (0,)), ((), ())),
                        preferred_element_type=jnp.float32) + bv
    odd_all = jnp.concatenate([r[0:1], r, r[-1:]], axis=0)
    for yy in range(_YB):
        ev = r[yy * w:(yy + 1) * w].reshape(w // 2, 2, 2, 128)
        od = odd_all[yy * w:yy * w + w + 2].reshape(np_, 2, 2, 128)
        for hg in range(2):
            out_ref[0, hg, yy, :] = od[:, :, hg, :]
            out_ref[1, hg, yy, 0:w // 2] = ev[:, :, hg, :]
            out_ref[1, hg, yy, w // 2] = ev[w // 2 - 1, :, hg, :]


def _value_project(bev3, wv, bv, h, w):
    c = bev3.shape[0]
    np_ = w // 2 + 1
    grid = (h // _YB,)
    return pl.pallas_call(
        functools.partial(_value_body, w=w),
        grid=grid,
        in_specs=[
            pl.BlockSpec((c, _YB, w), lambda i: (0, i, 0)),
            pl.BlockSpec((c, c), lambda i: (0, 0)),
            pl.BlockSpec((1, c), lambda i: (0, 0)),
        ],
        out_specs=pl.BlockSpec((2, 2, _YB, np_, 2, 128),
                               lambda i: (0, 0, i, 0, 0, 0)),
        out_shape=jax.ShapeDtypeStruct((2, 2, h, np_, 2, 128), jnp.float32),
    )(bev3, wv, bv)


def _stage_c_body(ms_ref, q_ref, wmo_ref, bmo_ref, wo_ref, bo_ref, out_ref):
    f32 = jnp.float32
    bf16 = jnp.bfloat16
    h1 = jnp.dot(ms_ref[...].astype(bf16), wmo_ref[...].astype(bf16),
                 preferred_element_type=f32)
    h1 = h1 + bmo_ref[...] + q_ref[...]
    out_ref[...] = jnp.dot(h1.astype(bf16), wo_ref[...].astype(bf16),
                           preferred_element_type=f32) + bo_ref[...]


def _stage_c(msda, q, wmo, bmo, wo, bo):
    nq, d = msda.shape
    grid = (nq // BQ,)
    return pl.pallas_call(
        _stage_c_body,
        grid=grid,
        in_specs=[
            pl.BlockSpec((BQ, d), lambda i: (i, 0)),
            pl.BlockSpec((BQ, d), lambda i: (i, 0)),
            pl.BlockSpec((d, d), lambda i: (0, 0)),
            pl.BlockSpec((1, d), lambda i: (0, 0)),
            pl.BlockSpec((d, d), lambda i: (0, 0)),
            pl.BlockSpec((1, d), lambda i: (0, 0)),
        ],
        out_specs=pl.BlockSpec((BQ, d), lambda i: (i, 0)),
        out_shape=jax.ShapeDtypeStruct((nq, d), jnp.float32),
    )(msda, q, wmo, bmo, wo, bo)


_NW = 32
_PPQ = 2 * HEADS * POINTS
_WPQ = 4 * HEADS * POINTS


def _sc_gather_combine(table, idx_flat, w_flat, nq):
    qpw = nq // _NW
    mesh = plsc.VectorSubcoreMesh(
        core_axis_name="c", subcore_axis_name="s", num_cores=2, num_subcores=16)

    chq = 2
    chr_ = chq * _PPQ
    nch = qpw // chq

    @functools.partial(
        pl.kernel,
        out_type=jax.ShapeDtypeStruct((nq, HEADS * HD), jnp.float32),
        mesh=mesh,
        scratch_types=[
            pltpu.VMEM((qpw * _PPQ,), jnp.int32),
            pltpu.VMEM((qpw * _WPQ,), jnp.float32),
            pltpu.VMEM((chr_, 2, 128), jnp.float32),
            pltpu.VMEM((chr_, 2, 128), jnp.float32),
            pltpu.VMEM((qpw, HEADS * HD), jnp.float32),
            pltpu.SemaphoreType.DMA,
            pltpu.SemaphoreType.DMA,
        ],
    )
    def k(table_hbm, idx_hbm, w_hbm, out_hbm, idx_v, w_v, rows0_v, rows1_v,
          out_v, sem0, sem1):
        wid = lax.axis_index("s") * 2 + lax.axis_index("c")
        pltpu.sync_copy(idx_hbm.at[pl.ds(wid * qpw * _PPQ, qpw * _PPQ)], idx_v)
        pltpu.sync_copy(w_hbm.at[pl.ds(wid * qpw * _WPQ, qpw * _WPQ)], w_v)

        def start(ch, buf, sem):
            pltpu.async_copy(
                table_hbm.at[idx_v.at[pl.ds(ch * chr_, chr_)]], buf, sem)

        def waitbuf(buf, sem):
            pltpu.make_async_copy(table_hbm.at[pl.ds(0, chr_)], buf, sem).wait()

        def combine(ch, buf):
            for qq in range(chq):
                wb = (ch * chq + qq) * _WPQ
                wvs = [w_v[pl.ds(wb + k * 16, 16)] for k in range(_WPQ // 16)]
                for h in range(HEADS):
                    acc0 = jnp.zeros((16,), jnp.float32)
                    acc1 = jnp.zeros((16,), jnp.float32)
                    ho = (h % 4) * HD
                    for c2 in range(2):
                        for p in range(POINTS):
                            pr = qq * _PPQ + c2 * HEADS * POINTS + h * POINTS + p
                            for slot in range(2):
                                rw = (c2 * 2 + slot) * HEADS * POINTS + h * POINTS + p
                                wgt = wvs[rw // 16][rw % 16]
                                acc0 = acc0 + wgt * buf[pr, slot, pl.ds(ho, 16)]
                                acc1 = acc1 + wgt * buf[pr, slot, pl.ds(ho + 16, 16)]
                    o = ch * chq + qq
                    out_v[o, pl.ds(h * HD, 16)] = acc0
                    out_v[o, pl.ds(h * HD + 16, 16)] = acc1

        start(0, rows0_v, sem0)

        def body(g, carry):
            start(2 * g + 1, rows1_v, sem1)
            waitbuf(rows0_v, sem0)
            combine(2 * g, rows0_v)

            @pl.when(g < nch // 2 - 1)
            def _():
                start(2 * g + 2, rows0_v, sem0)

            waitbuf(rows1_v, sem1)
            combine(2 * g + 1, rows1_v)
            return carry

        lax.fori_loop(0, nch // 2, body, 0)
        pltpu.sync_copy(out_v, out_hbm.at[pl.ds(wid * qpw, qpw)])

    return k(table, idx_flat, w_flat)


def kernel(query_embed, ctrl_points, bev_features, pc_range, Wq, bq, Wso, bso,
           Waw, baw, Wv, bv, Wmo, bmo, Wo, bo, spatial_shapes):
    b, nq, d = query_embed.shape
    _, c, h, w = bev_features.shape
    qe = query_embed.reshape(b * nq, d)
    ctrl8 = ctrl_points.reshape(b * nq, 8)
    wso_r = Wso.reshape(d, HEADS, POINTS, 2).transpose(0, 3, 1, 2).reshape(d, 2 * HEADS * POINTS)
    bso_r = bso.reshape(HEADS, POINTS, 2).transpose(2, 0, 1).reshape(1, 2 * HEADS * POINTS)

    qp, idx128, w128 = _stage_a(
        qe, ctrl8, Wq, bq.reshape(1, d), wso_r, bso_r,
        Waw, baw.reshape(1, HEADS * POINTS), pc_range, spatial_shapes)

    value = _value_project(bev_features.reshape(c, h, w), Wv, bv.reshape(1, d), h, w)
    table = value.reshape(2 * 2 * h * (w // 2 + 1), 2, 128)

    msda = _sc_gather_combine(table, idx128.reshape(-1), w128.reshape(-1), b * nq)

    out = _stage_c(msda, qp, Wmo, bmo.reshape(1, d),
                   Wo, bo.reshape(1, d))
    return out.reshape(b, nq, d)
